# baseline (speedup 1.0000x reference)
import functools

import jax
import jax.numpy as jnp
from jax import lax
from jax.experimental import pallas as pl
from jax.experimental.pallas import tpu as pltpu
from jax.experimental.pallas import tpu_sc as plsc

_NUM_CLASSES = 1000
_HIDDEN = 128
_NC = 2
_NS = 16


@functools.lru_cache(maxsize=None)
def _make_gather(B: int, D: int, V: int):
    NW = _NC * _NS
    assert B % NW == 0
    b_per_w = B // NW
    mesh = plsc.VectorSubcoreMesh(core_axis_name="c", subcore_axis_name="s")

    NCH = 8
    assert b_per_w % NCH == 0
    chsz = b_per_w // NCH

    @functools.partial(
        pl.kernel,
        mesh=mesh,
        out_type=jax.ShapeDtypeStruct((B, D), jnp.float32),
        scratch_types=[
            pltpu.VMEM((b_per_w,), jnp.int32),
            pltpu.VMEM((b_per_w, D), jnp.float32),
            pltpu.VMEM_SHARED((V, D), jnp.float32),
            pltpu.SemaphoreType.DMA,
            pltpu.SemaphoreType.DMA,
        ],
    )
    def gather_kernel(idx_hbm, table_hbm, out_hbm, idx_v, rows_v, table_sh,
                      sem_g, sem_w):
        sid = lax.axis_index("s")
        wid = sid * _NC + lax.axis_index("c")
        base = wid * b_per_w
        rows_a = 64
        @pl.when(sid < _NS - 1)
        def _():
            pltpu.sync_copy(
                table_hbm.at[pl.ds(sid * rows_a, rows_a)],
                table_sh.at[pl.ds(sid * rows_a, rows_a)],
            )

        @pl.when(sid == _NS - 1)
        def _():
            tail = (_NS - 1) * rows_a
            pltpu.sync_copy(
                table_hbm.at[pl.ds(tail, V - tail)],
                table_sh.at[pl.ds(tail, V - tail)],
            )

        pltpu.sync_copy(idx_hbm.at[pl.ds(base, b_per_w)], idx_v)
        plsc.subcore_barrier()
        gathers = [
            pltpu.async_copy(
                table_sh.at[idx_v.at[pl.ds(c * chsz, chsz)]],
                rows_v.at[pl.ds(c * chsz, chsz)],
                sem_g,
            )
            for c in range(NCH)
        ]
        writes = []
        for c in range(NCH):
            gathers[c].wait()
            writes.append(
                pltpu.async_copy(
                    rows_v.at[pl.ds(c * chsz, chsz)],
                    out_hbm.at[pl.ds(base + c * chsz, chsz)],
                    sem_w,
                )
            )
        for w in writes:
            w.wait()

    return gather_kernel


def kernel(labels, train, table):
    del train
    idx = labels.astype(jnp.int32)
    return _make_gather(labels.shape[0], table.shape[1], table.shape[0])(idx, table)


# speedup vs baseline: 1.0049x; 1.0049x over previous
import functools

import jax
import jax.numpy as jnp
from jax import lax
from jax.experimental import pallas as pl
from jax.experimental.pallas import tpu as pltpu
from jax.experimental.pallas import tpu_sc as plsc

_NUM_CLASSES = 1000
_HIDDEN = 128
_NC = 2
_NS = 16


@functools.lru_cache(maxsize=None)
def _make_gather(B: int, D: int, V: int):
    NW = _NC * _NS
    assert B % NW == 0
    b_per_w = B // NW
    mesh = plsc.VectorSubcoreMesh(core_axis_name="c", subcore_axis_name="s")

    NCH = 8
    assert b_per_w % NCH == 0
    chsz = b_per_w // NCH

    @functools.partial(
        pl.kernel,
        mesh=mesh,
        out_type=jax.ShapeDtypeStruct((B, D), jnp.float32),
        scratch_types=[
            pltpu.VMEM((b_per_w,), jnp.int32),
            pltpu.VMEM((b_per_w, D), jnp.float32),
            pltpu.VMEM_SHARED((V, D), jnp.float32),
            pltpu.SemaphoreType.DMA,
            pltpu.SemaphoreType.DMA,
        ],
    )
    def gather_kernel(idx_hbm, table_hbm, out_hbm, idx_v, rows_v, table_sh,
                      sem_g, sem_w):
        sid = lax.axis_index("s")
        wid = sid * _NC + lax.axis_index("c")
        base = wid * b_per_w
        half = (V // 2) & ~7
        @pl.when(sid == 0)
        def _():
            pltpu.sync_copy(
                table_hbm.at[pl.ds(0, half)], table_sh.at[pl.ds(0, half)]
            )

        @pl.when(sid == 1)
        def _():
            pltpu.sync_copy(
                table_hbm.at[pl.ds(half, V - half)],
                table_sh.at[pl.ds(half, V - half)],
            )

        pltpu.sync_copy(idx_hbm.at[pl.ds(base, b_per_w)], idx_v)
        plsc.subcore_barrier()
        gathers = [
            pltpu.async_copy(
                table_sh.at[idx_v.at[pl.ds(c * chsz, chsz)]],
                rows_v.at[pl.ds(c * chsz, chsz)],
                sem_g,
            )
            for c in range(NCH)
        ]
        writes = []
        for c in range(NCH):
            gathers[c].wait()
            writes.append(
                pltpu.async_copy(
                    rows_v.at[pl.ds(c * chsz, chsz)],
                    out_hbm.at[pl.ds(base + c * chsz, chsz)],
                    sem_w,
                )
            )
        for w in writes:
            w.wait()

    return gather_kernel


def kernel(labels, train, table):
    del train
    idx = labels.astype(jnp.int32)
    return _make_gather(labels.shape[0], table.shape[1], table.shape[0])(idx, table)


# speedup vs baseline: 1.0118x; 1.0068x over previous
import functools

import jax
import jax.numpy as jnp
from jax import lax
from jax.experimental import pallas as pl
from jax.experimental.pallas import tpu as pltpu
from jax.experimental.pallas import tpu_sc as plsc

_NUM_CLASSES = 1000
_HIDDEN = 128
_NC = 2
_NS = 16


@functools.lru_cache(maxsize=None)
def _make_gather(B: int, D: int, V: int):
    NW = _NC * _NS
    assert B % NW == 0
    b_per_w = B // NW
    mesh = plsc.VectorSubcoreMesh(core_axis_name="c", subcore_axis_name="s")

    NCH = 8
    assert b_per_w % NCH == 0
    chsz = b_per_w // NCH

    @functools.partial(
        pl.kernel,
        mesh=mesh,
        out_type=jax.ShapeDtypeStruct((B, D), jnp.float32),
        scratch_types=[
            pltpu.VMEM((b_per_w,), jnp.int32),
            pltpu.VMEM((b_per_w, D), jnp.float32),
            pltpu.VMEM_SHARED((V, D), jnp.float32),
            pltpu.SemaphoreType.DMA,
            pltpu.SemaphoreType.DMA,
        ],
    )
    def gather_kernel(idx_hbm, table_hbm, out_hbm, idx_v, rows_v, table_sh,
                      sem_g, sem_w):
        sid = lax.axis_index("s")
        wid = sid * _NC + lax.axis_index("c")
        base = wid * b_per_w
        @pl.when(sid == 0)
        def _():
            pltpu.sync_copy(table_hbm, table_sh)

        pltpu.sync_copy(idx_hbm.at[pl.ds(base, b_per_w)], idx_v)
        plsc.subcore_barrier()
        gathers = [
            pltpu.async_copy(
                table_sh.at[idx_v.at[pl.ds(c * chsz, chsz)]],
                rows_v.at[pl.ds(c * chsz, chsz)],
                sem_g,
            )
            for c in range(NCH)
        ]
        writes = []
        for c in range(NCH):
            gathers[c].wait()
            writes.append(
                pltpu.async_copy(
                    rows_v.at[pl.ds(c * chsz, chsz)],
                    out_hbm.at[pl.ds(base + c * chsz, chsz)],
                    sem_w,
                )
            )
        for w in writes:
            w.wait()

    return gather_kernel


def kernel(labels, train, table):
    del train
    idx = labels.astype(jnp.int32)
    return _make_gather(labels.shape[0], table.shape[1], table.shape[0])(idx, table)
